# Initial kernel scaffold; baseline (speedup 1.0000x reference)
import jax
import jax.numpy as jnp
from jax.experimental import pallas as pl


def kernel(y_pred, y_real, proj_mat):
    raise NotImplementedError("write your pallas kernel here")


# speedup vs baseline: 2.4074x; 2.4074x over previous
import jax
import jax.numpy as jnp
from jax.experimental import pallas as pl
from jax.experimental.pallas import tpu as pltpu

_B = 4
_N = 2048
_F = 1024
_P = 128


def _cmpex(x, j, k, iota):
    up = (iota & j) == 0
    desc = (iota & k) != 0
    pu = jnp.concatenate([x[j:], x[:j]], axis=0)
    pd = jnp.concatenate([x[_N - j:], x[: _N - j]], axis=0)
    p = jnp.where(up, pu, pd)
    keep_min = up != desc
    return jnp.where(keep_min, jnp.minimum(x, p), jnp.maximum(x, p))


def _bitonic_sort_cols(x):
    iota = jax.lax.broadcasted_iota(jnp.int32, (_N, 1), 0)
    k = 2
    while k <= _N:
        j = k // 2
        while j >= 1:
            x = _cmpex(x, j, k, iota)
            j //= 2
        k *= 2
    return x


def _swd_kernel(y_pred_ref, y_real_ref, proj_ref, out_ref):
    b = pl.program_id(0)
    proj = proj_ref[...]
    zp = jnp.dot(y_pred_ref[0], proj, preferred_element_type=jnp.float32)
    zr = jnp.dot(y_real_ref[0], proj, preferred_element_type=jnp.float32)
    z = jnp.concatenate([zp, zr], axis=1)
    z = _bitonic_sort_cols(z)
    d = z[:, :_P] - z[:, _P:]
    s = jnp.sum(d * d).reshape(1, 1)

    @pl.when(b == 0)
    def _():
        out_ref[...] = jnp.zeros((1, 1), jnp.float32)

    out_ref[...] += s


def kernel(y_pred, y_real, proj_mat):
    out = pl.pallas_call(
        _swd_kernel,
        grid=(_B,),
        in_specs=[
            pl.BlockSpec((1, _N, _F), lambda b: (b, 0, 0)),
            pl.BlockSpec((1, _N, _F), lambda b: (b, 0, 0)),
            pl.BlockSpec((_F, _P), lambda b: (0, 0)),
        ],
        out_specs=pl.BlockSpec((1, 1), lambda b: (0, 0)),
        out_shape=jax.ShapeDtypeStruct((1, 1), jnp.float32),
    )(y_pred, y_real, proj_mat)
    return (out / (_B * _N * _P)).reshape(())


# speedup vs baseline: 2.4253x; 1.0075x over previous
import jax
import jax.numpy as jnp
from jax.experimental import pallas as pl
from jax.experimental.pallas import tpu as pltpu

_B = 4
_N = 2048
_F = 1024
_P = 128


def _cmpex_rolled(x, j, up, keep_min):
    pu = jnp.concatenate([x[j:], x[:j]], axis=0)
    pd = jnp.concatenate([x[_N - j:], x[: _N - j]], axis=0)
    p = jnp.where(up, pu, pd)
    return jnp.where(keep_min, jnp.minimum(x, p), jnp.maximum(x, p))


def _cmpex_sliced(x, j, k):
    pieces = []
    for lo in range(0, _N, 2 * j):
        a = x[lo : lo + j]
        b = x[lo + j : lo + 2 * j]
        mn = jnp.minimum(a, b)
        mx = jnp.maximum(a, b)
        if lo & k:
            pieces += [mx, mn]
        else:
            pieces += [mn, mx]
    return jnp.concatenate(pieces, axis=0)


def _bitonic_sort_cols(x):
    iota = jax.lax.broadcasted_iota(jnp.int32, (_N, 1), 0)
    ups = {}
    bits = {}
    j = 1
    while j < _N:
        bits[j] = iota & j
        ups[j] = bits[j] == 0
        j *= 2
    k = 2
    while k <= _N:
        j = k // 2
        while j >= 1:
            if _N // (2 * j) <= 16:
                x = _cmpex_sliced(x, j, k)
            else:
                if k < _N:
                    keep_min = ups[j] != (bits[k] != 0)
                else:
                    keep_min = ups[j]
                x = _cmpex_rolled(x, j, ups[j], keep_min)
            j //= 2
        k *= 2
    return x


def _swd_kernel(y_pred_ref, y_real_ref, proj_ref, out_ref):
    b = pl.program_id(0)
    proj = proj_ref[...]
    zp = jnp.dot(y_pred_ref[0], proj, preferred_element_type=jnp.float32)
    zr = jnp.dot(y_real_ref[0], proj, preferred_element_type=jnp.float32)
    z = jnp.concatenate([zp, zr], axis=1)
    z = _bitonic_sort_cols(z)
    d = z[:, :_P] - z[:, _P:]
    s = jnp.sum(d * d).reshape(1, 1)

    @pl.when(b == 0)
    def _():
        out_ref[...] = jnp.zeros((1, 1), jnp.float32)

    out_ref[...] += s


def kernel(y_pred, y_real, proj_mat):
    out = pl.pallas_call(
        _swd_kernel,
        grid=(_B,),
        in_specs=[
            pl.BlockSpec((1, _N, _F), lambda b: (b, 0, 0)),
            pl.BlockSpec((1, _N, _F), lambda b: (b, 0, 0)),
            pl.BlockSpec((_F, _P), lambda b: (0, 0)),
        ],
        out_specs=pl.BlockSpec((1, 1), lambda b: (0, 0)),
        out_shape=jax.ShapeDtypeStruct((1, 1), jnp.float32),
    )(y_pred, y_real, proj_mat)
    return (out / (_B * _N * _P)).reshape(())
